# Initial kernel scaffold; baseline (speedup 1.0000x reference)
import jax
import jax.numpy as jnp
from jax.experimental import pallas as pl


def kernel(seq, seg, tok_table, seg_table):
    raise NotImplementedError("write your pallas kernel here")


# speedup vs baseline: 4.6337x; 4.6337x over previous
import functools

import jax
import jax.numpy as jnp
from jax import lax
from jax.experimental import pallas as pl
from jax.experimental.pallas import tpu as pltpu
from jax.experimental.pallas import tpu_sc as plsc

_VOCAB = 100000
_NSEG = 2
_LEN = 200
_D = 64
_B = 4096
_N = _B * _LEN

_NC, _NS, _L = 2, 16, 16
_NW = _NC * _NS
_NPW = _N // _NW
_C = 128
_CHUNKS = _NPW // _C


def _sinusoidal_encoding():
    position = jnp.arange(0, _LEN, dtype=jnp.float32)[:, None]
    div_term = jnp.exp(
        jnp.arange(0, _D, 2, dtype=jnp.float32)
        * -(jnp.log(jnp.asarray(10000.0)) / _D)
    )
    enc = jnp.zeros((_LEN, _D), dtype=jnp.float32)
    enc = enc.at[:, 0::2].set(jnp.sin(position * div_term))
    enc = enc.at[:, 1::2].set(jnp.cos(position * div_term))
    return enc


def _body(seq_hbm, seg_hbm, tok_hbm, add_hbm, out_hbm,
          seq_idx, seg_v, aidx, tok_buf, out_buf, sem1, sem2):
    wid = lax.axis_index("s") * _NC + lax.axis_index("c")
    wbase = wid * _NPW
    lanes = lax.iota(jnp.int32, _L)

    def chunk_body(c, carry):
        base = wbase + c * _C
        pltpu.sync_copy(seq_hbm.at[pl.ds(base, _C)], seq_idx)
        pltpu.sync_copy(seg_hbm.at[pl.ds(base, _C)], seg_v)
        for j in range(_C // _L):
            pos = lax.rem(base + j * _L + lanes, _LEN)
            aidx[pl.ds(j * _L, _L)] = seg_v[pl.ds(j * _L, _L)] * _LEN + pos
        cp1 = pltpu.async_copy(tok_hbm.at[seq_idx], tok_buf, sem1)
        cp2 = pltpu.async_copy(add_hbm.at[aidx], out_buf, sem2)
        cp1.wait()
        cp2.wait()

        def row(i, rcarry):
            for j in range(_D // _L):
                sl = pl.ds(j * _L, _L)
                t = tok_buf[i, sl]
                o = out_buf[i, sl]
                out_buf[i, sl] = o + t + t
            return rcarry

        lax.fori_loop(0, _C, row, 0)
        pltpu.sync_copy(out_buf, out_hbm.at[pl.ds(base, _C)])
        return carry

    lax.fori_loop(0, _CHUNKS, chunk_body, 0)


_sc_call = pl.kernel(
    _body,
    out_type=jax.ShapeDtypeStruct((_N, _D), jnp.float32),
    mesh=plsc.VectorSubcoreMesh(core_axis_name="c", subcore_axis_name="s"),
    scratch_types=[
        pltpu.VMEM((_C,), jnp.int32),
        pltpu.VMEM((_C,), jnp.int32),
        pltpu.VMEM((_C,), jnp.int32),
        pltpu.VMEM((_C, _D), jnp.float32),
        pltpu.VMEM((_C, _D), jnp.float32),
        pltpu.SemaphoreType.DMA,
        pltpu.SemaphoreType.DMA,
    ],
    compiler_params=pltpu.CompilerParams(use_tc_tiling_on_sc=False),
)


@jax.jit
def kernel(seq, seg, tok_table, seg_table):
    enc = _sinusoidal_encoding()
    addend = (seg_table[:, None, :] + enc[None, :, :]).reshape(
        _NSEG * _LEN, _D)
    seq_f = seq.reshape(_N).astype(jnp.int32)
    seg_f = seg.reshape(_N).astype(jnp.int32)
    out = _sc_call(seq_f, seg_f, tok_table, addend)
    return out.reshape(_B, _LEN, _D)


# speedup vs baseline: 5.5148x; 1.1902x over previous
import jax
import jax.numpy as jnp
from jax import lax
from jax.experimental import pallas as pl
from jax.experimental.pallas import tpu as pltpu
from jax.experimental.pallas import tpu_sc as plsc

_VOCAB = 100000
_NSEG = 2
_LEN = 200
_D = 64
_B = 4096
_N = _B * _LEN

_NC, _NS, _L = 2, 16, 16
_NW = _NC * _NS
_NPW = _N // _NW
_C = 128
_CHUNKS = _NPW // _C
_HALF = _CHUNKS // 2


def _sinusoidal_encoding():
    position = jnp.arange(0, _LEN, dtype=jnp.float32)[:, None]
    div_term = jnp.exp(
        jnp.arange(0, _D, 2, dtype=jnp.float32)
        * -(jnp.log(jnp.asarray(10000.0)) / _D)
    )
    enc = jnp.zeros((_LEN, _D), dtype=jnp.float32)
    enc = enc.at[:, 0::2].set(jnp.sin(position * div_term))
    enc = enc.at[:, 1::2].set(jnp.cos(position * div_term))
    return enc


def _body(seq_hbm, seg_hbm, tok_hbm, add_hbm, out_hbm,
          seq_idx0, aidx0, tok0, add0, wb0,
          seq_idx1, aidx1, tok1, add1, wb1,
          st0, sa0, sw0, st1, sa1, sw1):
    sets = (
        (seq_idx0, aidx0, tok0, add0, wb0, st0, sa0, sw0),
        (seq_idx1, aidx1, tok1, add1, wb1, st1, sa1, sw1),
    )
    wid = lax.axis_index("s") * _NC + lax.axis_index("c")
    wbase = wid * _NPW
    lanes = lax.iota(jnp.int32, _L)

    def stage(cc, bufs):
        seq_idx, aidx, tok_buf, add_buf, _, sem_tok, sem_add, _ = bufs
        base = wbase + cc * _C
        pltpu.sync_copy(seq_hbm.at[pl.ds(base, _C)], seq_idx)
        pltpu.sync_copy(seg_hbm.at[pl.ds(base, _C)], aidx)
        for j in range(_C // _L):
            sl = pl.ds(j * _L, _L)
            pos = lax.rem(base + j * _L + lanes, _LEN)
            aidx[sl] = aidx[sl] * _LEN + pos
        pltpu.async_copy(tok_hbm.at[seq_idx], tok_buf, sem_tok)
        pltpu.async_copy(add_hbm.at[aidx], add_buf, sem_add)

    stage(0, sets[0])
    stage(1, sets[1])

    def pair(i, carry):
        for s in range(2):
            bufs = sets[s]
            seq_idx, aidx, tok_buf, add_buf, wb_buf, sem_tok, sem_add, sem_wb = bufs
            cc = 2 * i + s
            base = wbase + cc * _C
            pltpu.make_async_copy(tok_hbm.at[seq_idx], tok_buf, sem_tok).wait()
            pltpu.make_async_copy(add_hbm.at[aidx], add_buf, sem_add).wait()

            @pl.when(i >= 1)
            def _wait_wb():
                pltpu.make_async_copy(
                    wb_buf, out_hbm.at[pl.ds(base, _C)], sem_wb).wait()

            @plsc.parallel_loop(0, _C, 1, unroll=4)
            def _combine(r):
                for j in range(_D // _L):
                    sl = pl.ds(j * _L, _L)
                    t = tok_buf[r, sl]
                    a = add_buf[r, sl]
                    wb_buf[r, sl] = a + t + t

            pltpu.async_copy(wb_buf, out_hbm.at[pl.ds(base, _C)], sem_wb)

            @pl.when(i < _HALF - 1)
            def _prefetch():
                stage(cc + 2, bufs)

        return carry

    lax.fori_loop(0, _HALF, pair, 0)
    for s in range(2):
        wb_buf, sem_wb = sets[s][4], sets[s][7]
        pltpu.make_async_copy(
            wb_buf, out_hbm.at[pl.ds(wbase, _C)], sem_wb).wait()


_sc_call = pl.kernel(
    _body,
    out_type=jax.ShapeDtypeStruct((_N, _D), jnp.float32),
    mesh=plsc.VectorSubcoreMesh(core_axis_name="c", subcore_axis_name="s"),
    scratch_types=[
        pltpu.VMEM((_C,), jnp.int32),
        pltpu.VMEM((_C,), jnp.int32),
        pltpu.VMEM((_C, _D), jnp.float32),
        pltpu.VMEM((_C, _D), jnp.float32),
        pltpu.VMEM((_C, _D), jnp.float32),
        pltpu.VMEM((_C,), jnp.int32),
        pltpu.VMEM((_C,), jnp.int32),
        pltpu.VMEM((_C, _D), jnp.float32),
        pltpu.VMEM((_C, _D), jnp.float32),
        pltpu.VMEM((_C, _D), jnp.float32),
        pltpu.SemaphoreType.DMA,
        pltpu.SemaphoreType.DMA,
        pltpu.SemaphoreType.DMA,
        pltpu.SemaphoreType.DMA,
        pltpu.SemaphoreType.DMA,
        pltpu.SemaphoreType.DMA,
    ],
    compiler_params=pltpu.CompilerParams(use_tc_tiling_on_sc=False),
)


@jax.jit
def kernel(seq, seg, tok_table, seg_table):
    enc = _sinusoidal_encoding()
    addend = (seg_table[:, None, :] + enc[None, :, :]).reshape(
        _NSEG * _LEN, _D)
    seq_f = seq.reshape(_N).astype(jnp.int32)
    seg_f = seg.reshape(_N).astype(jnp.int32)
    out = _sc_call(seq_f, seg_f, tok_table, addend)
    return out.reshape(_B, _LEN, _D)
